# baseline (speedup 1.0000x reference)
import jax
import jax.numpy as jnp
from jax import lax
from jax.experimental import pallas as pl
from jax.experimental.pallas import tpu as pltpu
from jax.experimental.pallas import tpu_sc as plsc

NC, NS = 2, 16
NW = NC * NS
LANE = 16
CHUNK = 128
IB = 16


def _sc_scatter_add(table, idx3, npad, rt):
    mesh = plsc.VectorSubcoreMesh(
        core_axis_name="c", subcore_axis_name="s",
        num_cores=NC, num_subcores=NS)
    zrows = npad // NS
    nblk = rt // IB

    def body(table_hbm, idx_hbm, out_hbm,
             ibuf, vals, acc, isem,
             gsem0, gsem1, gsem2, gsem3, gsem4, gsem5, gsem6, gsem7,
             ssem0, ssem1, ssem2, ssem3, ssem4, ssem5, ssem6, ssem7):
        c = lax.axis_index("c")
        s = lax.axis_index("s")
        wid = c * NS + s
        def zv(k, carry):
            vals[0, k, :] = jnp.zeros((LANE,), jnp.float32)
            return carry
        lax.fori_loop(0, CHUNK, zv, 0)
        def zcp(i, carry):
            pltpu.sync_copy(vals.at[0],
                            acc.at[pl.ds(s * zrows + i * CHUNK, CHUNK)])
            return carry
        lax.fori_loop(0, zrows // CHUNK, zcp, 0)
        plsc.subcore_barrier()

        base = wid * rt
        gsems = [gsem0, gsem1, gsem2, gsem3, gsem4, gsem5, gsem6, gsem7]
        ssems = [ssem0, ssem1, ssem2, ssem3, ssem4, ssem5, ssem6, ssem7]
        ND = 8

        pltpu.async_copy(idx_hbm.at[:, pl.ds(base, IB)], ibuf.at[0],
                         isem).wait()

        def outer(ob, carry):
            pb = lax.rem(ob, 2)
            cur = ibuf.at[pb]
            nxt = ibuf.at[1 - pb]

            nob = jnp.minimum(ob + 1, nblk - 1)
            nd = pltpu.async_copy(
                idx_hbm.at[:, pl.ds(base + nob * IB, IB)], nxt, isem)

            LOOKAHEAD = ND - 1
            descs_g = [None] * IB
            descs_s = [None] * IB
            for j in range(LOOKAHEAD):
                descs_g[j] = pltpu.async_copy(
                    table_hbm.at[cur.at[1, j]], vals.at[j % ND], gsems[j % ND])
            for j in range(IB):
                b = j % ND
                if j >= 1:
                    descs_s[j - 1].wait()
                if j + LOOKAHEAD < IB:
                    descs_g[j + LOOKAHEAD] = pltpu.async_copy(
                        table_hbm.at[cur.at[1, j + LOOKAHEAD]],
                        vals.at[(j + LOOKAHEAD) % ND],
                        gsems[(j + LOOKAHEAD) % ND])
                descs_g[j].wait()
                descs_s[j] = pltpu.async_copy(
                    vals.at[b], acc.at[cur.at[0, j]], ssems[b], add=True)
            descs_s[IB - 1].wait()
            nd.wait()
            return carry

        lax.fori_loop(0, nblk, outer, 0)
        plsc.subcore_barrier()
        pltpu.sync_copy(acc.at[pl.ds(s * zrows, zrows)],
                        out_hbm.at[c, pl.ds(s * zrows, zrows)])

    f = pl.kernel(
        body,
        out_type=jax.ShapeDtypeStruct((NC, npad, LANE), jnp.float32),
        mesh=mesh,
        scratch_types=[
            pltpu.VMEM((2, 2, IB, CHUNK), jnp.int32),
            pltpu.VMEM((8, CHUNK, LANE), jnp.float32),
            pltpu.VMEM_SHARED((npad, LANE), jnp.float32),
            pltpu.SemaphoreType.DMA,
            pltpu.SemaphoreType.DMA,
            pltpu.SemaphoreType.DMA,
            pltpu.SemaphoreType.DMA,
            pltpu.SemaphoreType.DMA,
            pltpu.SemaphoreType.DMA,
            pltpu.SemaphoreType.DMA,
            pltpu.SemaphoreType.DMA,
            pltpu.SemaphoreType.DMA,
            pltpu.SemaphoreType.DMA,
            pltpu.SemaphoreType.DMA,
            pltpu.SemaphoreType.DMA,
            pltpu.SemaphoreType.DMA,
            pltpu.SemaphoreType.DMA,
            pltpu.SemaphoreType.DMA,
            pltpu.SemaphoreType.DMA,
            pltpu.SemaphoreType.DMA,
        ],
        compiler_params=pltpu.CompilerParams(use_tc_tiling_on_sc=False),
    )
    return f(table, idx3)


def _tc_normalize_t(partial, npad):
    blk = npad // 49

    def body(p_ref, o_ref):
        p = p_ref[...]
        ssum = p[0] + p[1]
        deg = jnp.maximum(ssum[:, 8:9], 1.0)
        o_ref[...] = (ssum[:, 0:8] / deg).T

    return pl.pallas_call(
        body,
        grid=(49,),
        in_specs=[pl.BlockSpec((NC, blk, LANE), lambda i: (0, i, 0))],
        out_specs=pl.BlockSpec((8, blk), lambda i: (0, i)),
        out_shape=jax.ShapeDtypeStruct((8, npad), jnp.float32),
    )(partial)


def _tc_mlp(x_bm, w1col, b1col, w2col, b2, npad):
    h = w1col.shape[0]

    def body(x_ref, w1_ref, b1_ref, w2_ref, b2_ref, o_ref):
        xv = x_ref[...][:, None, :]
        w1v = w1_ref[...][None, :, 0:1]
        b1v = b1_ref[...][None, :, 0:1]
        hid = xv * w1v + b1v
        z = jnp.abs(hid) * 0.7071067811865476
        t = 1.0 / (1.0 + 0.3275911 * z)
        poly = t * (0.254829592 + t * (-0.284496736 + t * (
            1.421413741 + t * (-1.453152027 + t * 1.061405429))))
        erfz = 1.0 - poly * jnp.exp(-z * z)
        serf = jnp.where(hid >= 0, erfz, -erfz)
        g = 0.5 * hid * (1.0 + serf)
        y = jnp.sum(g * w2_ref[...][None, :, 0:1], axis=1)
        o_ref[...] = y + b2_ref[0]

    return pl.pallas_call(
        body,
        grid=(npad // 128,),
        in_specs=[
            pl.BlockSpec((8, 128), lambda i: (0, i)),
            pl.BlockSpec((h, 1), lambda i: (0, 0)),
            pl.BlockSpec((h, 1), lambda i: (0, 0)),
            pl.BlockSpec((h, 1), lambda i: (0, 0)),
            pl.BlockSpec(memory_space=pltpu.SMEM),
        ],
        out_specs=pl.BlockSpec((8, 128), lambda i: (0, i)),
        out_shape=jax.ShapeDtypeStruct((8, npad), jnp.float32),
    )(x_bm, w1col, b1col, w2col, b2)


def kernel(mu, edge_index, W1, b1, W2, b2):
    B, N = mu.shape
    E = edge_index.shape[1]
    H = W1.shape[0]

    npad = 100352
    rt = 784
    e_pad = NW * rt * CHUNK - E

    mu_t = mu.T
    table = jnp.concatenate(
        [mu_t, jnp.ones((N, 1), jnp.float32),
         jnp.zeros((N, LANE - B - 1), jnp.float32)], axis=1)

    pad_idx = jnp.stack([jnp.full((e_pad,), N, jnp.int32),
                         jnp.zeros((e_pad,), jnp.int32)])
    idx3 = jnp.concatenate([edge_index, pad_idx], axis=1).reshape(
        2, NW * rt, CHUNK)
    partial = _sc_scatter_add(table, idx3, npad, rt)
    x_bm = _tc_normalize_t(partial, npad)
    y = _tc_mlp(x_bm, W1, b1.reshape(H, 1), W2.reshape(H, 1), b2, npad)
    return y[:, :N]
